# baseline (speedup 1.0000x reference)
import functools

import jax
import jax.numpy as jnp
from jax import lax
from jax.experimental import pallas as pl
from jax.experimental.pallas import tpu as pltpu
from jax.experimental.pallas import tpu_sc as plsc

GENE_NUM = 100000
D = 32
N = 16384
NC = 2
NS = 16
L = 16
NW = NC * NS
RPW = N // NW
CH = 64
NCH = RPW // CH
ORPW = RPW // 2


def _sc_body(we_hbm, be_hbm, wm_hbm, bm_hbm, pos_hbm, exp_hbm, flg_hbm,
             out_hbm,
             rowe_v, offe_v, rowm_v, offm_v, pos_v, exp_v, flg_v,
             we_v, be_v, wm_v, bm_v, out_v, sems):
    wid = lax.axis_index("s") * NC + lax.axis_index("c")
    base = wid * RPW

    pltpu.sync_copy(pos_hbm.at[pl.ds(base, RPW)], pos_v)
    pltpu.sync_copy(flg_hbm.at[pl.ds(base, RPW)], flg_v)
    pltpu.sync_copy(exp_hbm.at[pl.ds(base, RPW)], exp_v)

    def mk_idx(i, carry):
        sl = pl.ds(i * L, L)
        p = pos_v[sl]
        j2 = p * 2 + flg_v[sl].astype(jnp.int32)
        rowe_v[sl] = lax.shift_right_logical(p, 2)
        offe_v[sl] = lax.shift_left(jnp.bitwise_and(p, 3), 5)
        rowm_v[sl] = lax.shift_right_logical(j2, 2)
        offm_v[sl] = lax.shift_left(jnp.bitwise_and(j2, 3), 5)
        return carry
    lax.fori_loop(0, RPW // L, mk_idx, 0)

    def fire(c):
        p = c % 2
        s = pl.ds(c * CH, CH)
        return [
            pltpu.async_copy(we_hbm.at[rowe_v.at[s]], we_v.at[p], sems.at[0, p]),
            pltpu.async_copy(be_hbm.at[rowe_v.at[s]], be_v.at[p], sems.at[1, p]),
            pltpu.async_copy(wm_hbm.at[rowm_v.at[s]], wm_v.at[p], sems.at[2, p]),
            pltpu.async_copy(bm_hbm.at[rowe_v.at[s]], bm_v.at[p], sems.at[3, p]),
        ]

    def compute(c):
        p = c % 2
        cbase = c * CH

        def grp(g, carry):
            r0 = g * L
            sl = pl.ds(cbase + r0, L)
            ev = exp_v[sl]
            owv = offe_v[sl]
            omv = offm_v[sl]
            ob = lax.shift_right_logical(cbase + r0, 1)
            for j in range(L):
                r = r0 + j
                e = ev[j]
                ow = owv[j]
                om = omv[j]
                o2 = ob + j // 2
                par = (j % 2) * (4 * L)
                out_v[o2, pl.ds(par, L)] = (we_v[p, r, pl.ds(ow, L)] * e
                                            + be_v[p, r, pl.ds(ow, L)])
                out_v[o2, pl.ds(par + L, L)] = (we_v[p, r, pl.ds(ow + L, L)] * e
                                                + be_v[p, r, pl.ds(ow + L, L)])
                out_v[o2, pl.ds(par + 2 * L, L)] = (wm_v[p, r, pl.ds(om, L)]
                                                    + bm_v[p, r, pl.ds(ow, L)])
                out_v[o2, pl.ds(par + 3 * L, L)] = (wm_v[p, r, pl.ds(om + L, L)]
                                                    + bm_v[p, r, pl.ds(ow + L, L)])
            return carry
        lax.fori_loop(0, CH // L, grp, 0)

    pending = fire(0)
    for c in range(NCH):
        cur = pending
        if c + 1 < NCH:
            pending = fire(c + 1)
        for cp in cur:
            cp.wait()
        compute(c)

    pltpu.sync_copy(out_v, out_hbm.at[pl.ds(wid * ORPW, ORPW)])


_sc_kernel = functools.partial(
    pl.kernel,
    mesh=plsc.VectorSubcoreMesh(core_axis_name="c", subcore_axis_name="s"),
    out_type=jax.ShapeDtypeStruct((N // 2, 4 * D), jnp.float32),
    scratch_types=[
        pltpu.VMEM((RPW,), jnp.int32),
        pltpu.VMEM((RPW,), jnp.int32),
        pltpu.VMEM((RPW,), jnp.int32),
        pltpu.VMEM((RPW,), jnp.int32),
        pltpu.VMEM((RPW,), jnp.int32),
        pltpu.VMEM((RPW,), jnp.float32),
        pltpu.VMEM((RPW,), jnp.float32),
        pltpu.VMEM((2, CH, 128), jnp.float32),
        pltpu.VMEM((2, CH, 128), jnp.float32),
        pltpu.VMEM((2, CH, 128), jnp.float32),
        pltpu.VMEM((2, CH, 128), jnp.float32),
        pltpu.VMEM((ORPW, 4 * D), jnp.float32),
        pltpu.SemaphoreType.DMA((4, 2)),
    ],
)(_sc_body)


def kernel(x, pos, weight_exp, bias_exp, weight_mu, bias_mu):
    pos32 = pos.astype(jnp.int32)
    exp_col = x[:, 0]
    flg_col = x[:, 1]
    we4 = weight_exp.reshape(GENE_NUM // 4, 128)
    be4 = bias_exp.reshape(GENE_NUM // 4, 128)
    wm4 = weight_mu.reshape(GENE_NUM // 2, 128)
    bm4 = bias_mu.reshape(GENE_NUM // 4, 128)
    out = _sc_kernel(we4, be4, wm4, bm4, pos32, exp_col, flg_col)
    return out.reshape(N, 2 * D)
